# baseline (speedup 1.0000x reference)
import jax
import jax.numpy as jnp
from jax.experimental import pallas as pl
from jax.experimental.pallas import tpu as pltpu
from jax.experimental.pallas import tpu_sc as plsc

_CHUNKS_PER_CORE = 4


def kernel(x, scale_p, scale_n):
    del scale_p, scale_n
    m, n = x.shape
    mesh = plsc.ScalarSubcoreMesh(axis_name="core", num_cores=2)
    rows = m // (2 * _CHUNKS_PER_CORE)

    @pl.kernel(
        out_type=jax.ShapeDtypeStruct((m, n), x.dtype),
        mesh=mesh,
        scratch_types=[pltpu.SemaphoreType.DMA((_CHUNKS_PER_CORE,))],
    )
    def sc_copy(x_ref, o_ref, sems):
        core = jax.lax.axis_index("core")
        base = core * (m // 2)

        def copy(c):
            start = base + c * rows
            return pltpu.async_copy(
                x_ref.at[pl.ds(start, rows), :],
                o_ref.at[pl.ds(start, rows), :],
                sems.at[c],
            )

        for c in range(_CHUNKS_PER_CORE):
            copy(c).start()
        for c in range(_CHUNKS_PER_CORE):
            copy(c).wait()

    return sc_copy(x)
